# baseline (speedup 1.0000x reference)
import functools

import jax
import jax.numpy as jnp
from jax import lax
from jax.experimental import pallas as pl
from jax.experimental.pallas import tpu as pltpu
from jax.experimental.pallas import tpu_sc as plsc

_BASE = 10000
_NC = 2
_NS = 16
_NW = _NC * _NS
_L = 16


def _splat_lane(vec, lane):
    idx = jnp.full((_L, 1), lane, jnp.int32)
    return lax.gather(
        vec, idx,
        lax.GatherDimensionNumbers(
            offset_dims=(), collapsed_slice_dims=(0,), start_index_map=(0,)),
        slice_sizes=(1,),
        mode=lax.GatherScatterMode.PROMISE_IN_BOUNDS)


def _rope_coeffs(seq_len: int, dim: int):
    freqs = 1.0 / (_BASE ** (jnp.arange(0, dim, 2, dtype=jnp.float32) / dim))
    ang = jnp.outer(jnp.arange(seq_len, dtype=jnp.float32), freqs)
    cos = jnp.cos(ang)
    sin = jnp.sin(ang)
    a = jnp.stack([cos, cos], axis=-1).reshape(seq_len, dim)
    c = jnp.stack([sin, sin], axis=-1).reshape(seq_len, dim)
    return a, c


def _make_sc_kernel(bsz: int, seq_len: int, dim: int):
    assert bsz % (_NW * 128) == 0 or bsz // _NW == 128
    bpw = bsz // _NW
    assert bpw == 128 and dim % (2 * _L) == 0
    dh = dim // 8
    bh = bsz // 128
    mesh = plsc.VectorSubcoreMesh(core_axis_name="c", subcore_axis_name="s")

    @functools.partial(
        pl.kernel,
        mesh=mesh,
        out_type=jax.ShapeDtypeStruct((seq_len, dh, bh, 8, 128), jnp.float32),
        compiler_params=pltpu.CompilerParams(
            use_tc_tiling_on_sc=False, needs_layout_passes=False),
        scratch_types=[
            pltpu.VMEM((seq_len, dim), jnp.float32),
            pltpu.VMEM((seq_len, dim), jnp.float32),
            pltpu.VMEM((seq_len, bpw), jnp.int32),
            pltpu.VMEM((bpw, dim), jnp.float32),
            pltpu.VMEM((bpw, dim), jnp.float32),
            pltpu.VMEM((dh, 8, 128), jnp.float32),
            pltpu.VMEM((dh, 8, 128), jnp.float32),
            pltpu.SemaphoreType.DMA,
            pltpu.SemaphoreType.DMA,
            pltpu.SemaphoreType.DMA,
            pltpu.SemaphoreType.DMA,
        ],
    )
    def emb(table_hbm, ids_hbm, a_hbm, c_hbm, out_hbm,
            a_v, c_v, ids_v, x0, x1, o0, o1, gs0, gs1, ss0, ss1):
        wid = lax.axis_index("s") * _NC + lax.axis_index("c")
        pltpu.sync_copy(ids_hbm.at[wid], ids_v)
        pltpu.sync_copy(a_hbm, a_v)
        pltpu.sync_copy(c_hbm, c_v)
        iota = lax.iota(jnp.int32, _L)

        def start_gather(s, xbuf, sem):
            pltpu.async_copy(table_hbm.at[ids_v.at[s]], xbuf, sem)

        def wait_gather(s, xbuf, sem):
            pltpu.make_async_copy(table_hbm.at[ids_v.at[s]], xbuf, sem).wait()

        def start_store(s, obuf, sem):
            pltpu.async_copy(obuf, out_hbm.at[s, :, wid], sem)

        def wait_store(s, obuf, sem):
            pltpu.make_async_copy(obuf, out_hbm.at[s, :, wid], sem).wait()

        def compute(s, xbuf, obuf):
            @plsc.parallel_loop(0, dim // 2, 1, unroll=2)
            def pair_k(k):
                d = 2 * k
                ch = d & ~(_L - 1)
                ln = d & (_L - 1)
                a_ch = a_v[s, pl.ds(ch, _L)]
                c_ch = c_v[s, pl.ds(ch, _L)]
                cosv = _splat_lane(a_ch, ln)
                sinv = _splat_lane(c_ch, ln)
                de_h, de_l = d >> 3, d & 7
                cole = jnp.full((_L,), d, jnp.int32)
                colo = jnp.full((_L,), d + 1, jnp.int32)
                for c in range(bpw // _L):
                    rows = iota + (c * _L)
                    xe = plsc.load_gather(xbuf, [rows, cole])
                    xo = plsc.load_gather(xbuf, [rows, colo])
                    sl = pl.ds(c * _L, _L)
                    obuf[de_h, de_l, sl] = xe * cosv - xo * sinv
                    obuf[de_h, de_l + 1, sl] = xe * sinv + xo * cosv

        start_gather(0, x0, gs0)
        start_gather(1, x1, gs1)

        def half(qq, s, xbuf, obuf, gsem, ssem):
            wait_gather(s, xbuf, gsem)

            @pl.when(qq > 0)
            def _():
                wait_store(s - 2, obuf, ssem)

            compute(s, xbuf, obuf)
            start_store(s, obuf, ssem)

            @pl.when(s + 2 < seq_len)
            def _():
                start_gather(s + 2, xbuf, gsem)

        def pair_body(qq, carry):
            half(qq, 2 * qq, x0, o0, gs0, ss0)
            half(qq, 2 * qq + 1, x1, o1, gs1, ss1)
            return carry

        lax.fori_loop(0, seq_len // 2, pair_body, 0, unroll=False)
        wait_store(seq_len - 2, o0, ss0)
        wait_store(seq_len - 1, o1, ss1)

    return emb


def kernel(token_ids, table):
    bsz, seq_len = token_ids.shape
    vocab, dim = table.shape
    ids = token_ids.reshape(_NW, bsz // _NW, seq_len).transpose(0, 2, 1)
    ids = ids.astype(jnp.int32)
    a, c = _rope_coeffs(seq_len, dim)
    out5 = _make_sc_kernel(bsz, seq_len, dim)(table, ids, a, c)
    return out5.transpose(2, 4, 0, 1, 3).reshape(bsz, seq_len, dim)


# speedup vs baseline: 1.5916x; 1.5916x over previous
import functools

import jax
import jax.numpy as jnp
from jax import lax
from jax.experimental import pallas as pl
from jax.experimental.pallas import tpu as pltpu
from jax.experimental.pallas import tpu_sc as plsc

_BASE = 10000
_NC = 2
_NS = 16
_NW = _NC * _NS
_L = 16


def _lane_swap(x, idx2d):
    return lax.gather(
        x, idx2d,
        lax.GatherDimensionNumbers(
            offset_dims=(), collapsed_slice_dims=(0,), start_index_map=(0,)),
        slice_sizes=(1,),
        mode=lax.GatherScatterMode.PROMISE_IN_BOUNDS)


def _rope_coeffs(seq_len: int, dim: int):
    freqs = 1.0 / (_BASE ** (jnp.arange(0, dim, 2, dtype=jnp.float32) / dim))
    ang = jnp.outer(jnp.arange(seq_len, dtype=jnp.float32), freqs)
    cos = jnp.cos(ang)
    sin = jnp.sin(ang)
    a = jnp.stack([cos, cos], axis=-1).reshape(seq_len, dim)
    b = jnp.stack([-sin, sin], axis=-1).reshape(seq_len, dim)
    return a, b


def _make_sc_kernel(bsz: int, seq_len: int, dim: int):
    assert bsz % (_NW * 128) == 0 or bsz // _NW == 128
    bpw = bsz // _NW
    assert bpw == 128 and dim % (2 * _L) == 0
    dh = dim // 8
    bh = bsz // 128
    mesh = plsc.VectorSubcoreMesh(core_axis_name="c", subcore_axis_name="s")

    @functools.partial(
        pl.kernel,
        mesh=mesh,
        out_type=jax.ShapeDtypeStruct((seq_len, dh, bh, 8, 128), jnp.float32),
        compiler_params=pltpu.CompilerParams(
            use_tc_tiling_on_sc=False, needs_layout_passes=False),
        scratch_types=[
            pltpu.VMEM((seq_len, dim), jnp.float32),
            pltpu.VMEM((seq_len, dim), jnp.float32),
            pltpu.VMEM((seq_len, bpw), jnp.int32),
            pltpu.VMEM((bpw, dim), jnp.float32),
            pltpu.VMEM((bpw, dim), jnp.float32),
            pltpu.VMEM((dh, 8, 129), jnp.float32),
            pltpu.VMEM((dh, 8, 129), jnp.float32),
            pltpu.SemaphoreType.DMA,
            pltpu.SemaphoreType.DMA,
            pltpu.SemaphoreType.DMA,
            pltpu.SemaphoreType.DMA,
        ],
    )
    def emb(table_hbm, ids_hbm, a_hbm, b_hbm, out_hbm,
            a_v, b_v, ids_v, x0, x1, o0, o1, gs0, gs1, ss0, ss1):
        wid = lax.axis_index("s") * _NC + lax.axis_index("c")
        pltpu.sync_copy(ids_hbm.at[wid], ids_v)
        pltpu.sync_copy(a_hbm, a_v)
        pltpu.sync_copy(b_hbm, b_v)
        iota = lax.iota(jnp.int32, _L)

        def start_gather(s, xbuf, sem):
            pltpu.async_copy(table_hbm.at[ids_v.at[s]], xbuf, sem)

        def wait_gather(s, xbuf, sem):
            pltpu.make_async_copy(table_hbm.at[ids_v.at[s]], xbuf, sem).wait()

        def start_store(s, obuf, sem):
            pltpu.async_copy(
                obuf.at[:, :, pl.ds(0, 128)], out_hbm.at[s, :, wid], sem)

        def wait_store(s, obuf, sem):
            pltpu.make_async_copy(
                obuf.at[:, :, pl.ds(0, 128)], out_hbm.at[s, :, wid], sem).wait()

        swap2d = (iota ^ 1).reshape(_L, 1)
        dh_vecs = [((j * _L) + iota) >> 3 for j in range(dim // _L)]
        dl_vecs = [((j * _L) + iota) & 7 for j in range(dim // _L)]

        def compute(s, xbuf, obuf):
            avs = [a_v[s, pl.ds(j * _L, _L)] for j in range(dim // _L)]
            bvs = [b_v[s, pl.ds(j * _L, _L)] for j in range(dim // _L)]

            @plsc.parallel_loop(0, bpw, 1, unroll=4)
            def row_body(r):
                rfull = jnp.full((_L,), r, jnp.int32)
                for j in range(dim // _L):
                    x = xbuf[r, pl.ds(j * _L, _L)]
                    xsw = _lane_swap(x, swap2d)
                    val = x * avs[j] + xsw * bvs[j]
                    plsc.store_scatter(obuf, [dh_vecs[j], dl_vecs[j], rfull], val)

        start_gather(0, x0, gs0)
        start_gather(1, x1, gs1)

        def half(qq, s, xbuf, obuf, gsem, ssem):
            wait_gather(s, xbuf, gsem)

            @pl.when(qq > 0)
            def _():
                wait_store(s - 2, obuf, ssem)

            compute(s, xbuf, obuf)
            start_store(s, obuf, ssem)

            @pl.when(s + 2 < seq_len)
            def _():
                start_gather(s + 2, xbuf, gsem)

        def pair_body(qq, carry):
            half(qq, 2 * qq, x0, o0, gs0, ss0)
            half(qq, 2 * qq + 1, x1, o1, gs1, ss1)
            return carry

        lax.fori_loop(0, seq_len // 2, pair_body, 0, unroll=False)
        wait_store(seq_len - 2, o0, ss0)
        wait_store(seq_len - 1, o1, ss1)

    return emb


def kernel(token_ids, table):
    bsz, seq_len = token_ids.shape
    vocab, dim = table.shape
    ids = token_ids.reshape(_NW, bsz // _NW, seq_len).transpose(0, 2, 1)
    ids = ids.astype(jnp.int32)
    a, b = _rope_coeffs(seq_len, dim)
    out5 = _make_sc_kernel(bsz, seq_len, dim)(table, ids, a, b)
    return out5.transpose(2, 4, 0, 1, 3).reshape(bsz, seq_len, dim)
